# baseline (speedup 1.0000x reference)
import functools

import jax
import jax.numpy as jnp
from jax import lax
from jax.experimental import pallas as pl
from jax.experimental.pallas import tpu as pltpu
from jax.experimental.pallas import tpu_sc as plsc

_B, _L, _H, _P, _E = 1024, 50, 128, 512, 4
_N = _B * _L
_NC, _NS = 2, 16
_NW = _NC * _NS
_CH = 40
_S = 4
_NT = _N // _S
_T = 512


def _sc_gather(item_t, text_t, img_t, ids):
    n = ids.shape[0]
    pw = n // _NW
    nchunk = pw // _CH

    @functools.partial(
        pl.kernel,
        mesh=plsc.VectorSubcoreMesh(core_axis_name="c", subcore_axis_name="s"),
        out_type=(
            jax.ShapeDtypeStruct((n, _H), jnp.float32),
            jax.ShapeDtypeStruct((n, _P), jnp.float32),
            jax.ShapeDtypeStruct((n, _P), jnp.float32),
        ),
        scratch_types=(
            pltpu.VMEM((2, _CH), jnp.int32),
            pltpu.VMEM((2, _CH, _H), jnp.float32),
            pltpu.VMEM((2, _CH, _P), jnp.float32),
            pltpu.VMEM((2, _CH, _P), jnp.float32),
            pltpu.SemaphoreType.DMA,
            pltpu.SemaphoreType.DMA,
            pltpu.SemaphoreType.DMA,
            pltpu.SemaphoreType.DMA,
        ),
    )
    def gather_kernel(item_hbm, text_hbm, img_hbm, ids_hbm,
                      o_item, o_text, o_img,
                      idx_v, buf_h, buf_t, buf_i, g0, g1, w0, w1):
        wid = lax.axis_index("s") * _NC + lax.axis_index("c")
        gsem = (g0, g1)
        wsem = (w0, w1)
        gd = [None, None]
        wd = [None, None]

        def start(k):
            pr = k % 2
            if wd[pr] is not None:
                for d in wd[pr]:
                    d.wait()
                wd[pr] = None
            sl = pl.ds(wid * pw + k * _CH, _CH)
            pltpu.sync_copy(ids_hbm.at[sl], idx_v.at[pr])
            gd[pr] = (
                pltpu.async_copy(item_hbm.at[idx_v.at[pr]], buf_h.at[pr], gsem[pr]),
                pltpu.async_copy(text_hbm.at[idx_v.at[pr]], buf_t.at[pr], gsem[pr]),
                pltpu.async_copy(img_hbm.at[idx_v.at[pr]], buf_i.at[pr], gsem[pr]),
            )

        def finish(k):
            pr = k % 2
            for d in gd[pr]:
                d.wait()
            gd[pr] = None
            sl = pl.ds(wid * pw + k * _CH, _CH)
            wd[pr] = (
                pltpu.async_copy(buf_h.at[pr], o_item.at[sl], wsem[pr]),
                pltpu.async_copy(buf_t.at[pr], o_text.at[sl], wsem[pr]),
                pltpu.async_copy(buf_i.at[pr], o_img.at[sl], wsem[pr]),
            )

        start(0)
        for k in range(1, nchunk):
            start(k)
            finish(k - 1)
        finish(nchunk - 1)
        for pr in (0, 1):
            if wd[pr] is not None:
                for d in wd[pr]:
                    d.wait()

    return gather_kernel(item_t, text_t, img_t, ids)


def _dot(a, b):
    return lax.dot_general(a, b, (((1,), (0,)), ((), ())),
                           preferred_element_type=jnp.float32)


def _tc_body(text_r, img_r, item_r, nt_r, ni_r,
             ftw, ftb, fiw, fib,
             mtw, mtb, stw, stb, miw, mib, siw, sib,
             gw, gb, tew, teb, iew, ieb,
             fw, fb, fg, fbeta, out_r):
    def proj(x, w, b):
        y = _dot(x, w[...]) + b[...]
        nrm = jnp.sqrt(jnp.sum(y * y, axis=-1, keepdims=True))
        return y / jnp.maximum(nrm, 1e-12)

    te = proj(text_r[...], ftw, ftb)
    ie = proj(img_r[...], fiw, fib)

    t_z = _dot(te, mtw[...]) + mtb[...] + jnp.exp(_dot(te, stw[...]) + stb[...]) * nt_r[...]
    i_z = _dot(ie, miw[...]) + mib[...] + jnp.exp(_dot(ie, siw[...]) + sib[...]) * ni_r[...]

    jj = lax.broadcasted_iota(jnp.int32, (_E, _E * _H), 0)
    ll = lax.broadcasted_iota(jnp.int32, (_E, _E * _H), 1)
    ee = (jj == (ll >> 7)).astype(jnp.float32)
    neg = jnp.float32(-1e30)

    def moe(z, ewc, ebc):
        logits = _dot(z, gw[...]) + gb[...]
        lt = logits.T
        ii = lax.broadcasted_iota(jnp.int32, (_E, _T), 0)
        m1 = jnp.max(lt, axis=0, keepdims=True)
        a1 = jnp.min(jnp.where(lt == m1, ii, _E), axis=0, keepdims=True)
        msk = jnp.where(ii == a1, neg, lt)
        m2 = jnp.max(msk, axis=0, keepdims=True)
        a2 = jnp.min(jnp.where(msk == m2, ii, _E), axis=0, keepdims=True)
        keep = (ii == a1) | (ii == a2)
        e = jnp.exp(lt - m1)
        w = jnp.where(keep, e, 0.0)
        wn = w / jnp.sum(w, axis=0, keepdims=True)
        gx = lax.dot_general(wn, ee, (((0,), (0,)), ((), ())),
                             preferred_element_type=jnp.float32)
        y = (_dot(z, ewc[...]) + ebc[...]) * gx
        return (y[:, 0:_H] + y[:, _H:2 * _H]
                + y[:, 2 * _H:3 * _H] + y[:, 3 * _H:4 * _H])

    t_out = moe(t_z, tew, teb)
    i_out = moe(i_z, iew, ieb)

    f = _dot(t_out, fw[0]) + _dot(i_out, fw[1]) + fb[...]
    mu = jnp.mean(f, axis=-1, keepdims=True)
    d = f - mu
    v = jnp.mean(d * d, axis=-1, keepdims=True)
    ln = d / jnp.sqrt(v + 1e-5) * fg[...] + fbeta[...]
    out_r[...] = item_r[...] + jnp.maximum(ln, 0.0)


def _tc_specs_and_args(item_g, text_g, img_g, nt, ni, p):
    tok = lambda d: pl.BlockSpec((_T, d), lambda i: (i, 0))
    full = lambda *shape: pl.BlockSpec(shape, lambda i: (0,) * len(shape))
    r2 = lambda x: x.reshape(1, -1)
    args = (
        text_g, img_g, item_g, nt, ni,
        p["fc_text_w"], r2(p["fc_text_b"]), p["fc_img_w"], r2(p["fc_img_b"]),
        p["mu_t_w"], r2(p["mu_t_b"]), p["sg_t_w"], r2(p["sg_t_b"]),
        p["mu_i_w"], r2(p["mu_i_b"]), p["sg_i_w"], r2(p["sg_i_b"]),
        p["gate_w"], r2(p["gate_b"]),
        jnp.transpose(p["te_w"], (1, 0, 2)).reshape(_H, _E * _H),
        p["te_b"].reshape(1, _E * _H),
        jnp.transpose(p["ie_w"], (1, 0, 2)).reshape(_H, _E * _H),
        p["ie_b"].reshape(1, _E * _H),
        p["fus_w"].reshape(2, _H, _H), r2(p["fus_b"]),
        r2(p["fus_ln_g"]), r2(p["fus_ln_b"]),
    )
    in_specs = [
        tok(_P), tok(_P), tok(_H), tok(_H), tok(_H),
        full(_P, _H), full(1, _H), full(_P, _H), full(1, _H),
        full(_H, _H), full(1, _H), full(_H, _H), full(1, _H),
        full(_H, _H), full(1, _H), full(_H, _H), full(1, _H),
        full(_H, _E), full(1, _E),
        full(_H, _E * _H), full(1, _E * _H),
        full(_H, _E * _H), full(1, _E * _H),
        full(2, _H, _H), full(1, _H),
        full(1, _H), full(1, _H),
    ]
    return in_specs, args


def _tc_dense(item_g, text_g, img_g, nt, ni, p):
    n = item_g.shape[0]
    in_specs, args = _tc_specs_and_args(item_g, text_g, img_g, nt, ni, p)
    return pl.pallas_call(
        _tc_body,
        grid=(n // _T,),
        in_specs=in_specs,
        out_specs=pl.BlockSpec((_T, _H), lambda i: (i, 0)),
        out_shape=jax.ShapeDtypeStruct((n, _H), jnp.float32),
        compiler_params=pltpu.CompilerParams(
            dimension_semantics=("arbitrary",),
        ),
    )(*args)


def kernel(params, noise_t, noise_i, input_ids):
    p = params
    ids = input_ids.reshape(-1).astype(jnp.int32)
    nt = noise_t.reshape(_N, _H)
    ni = noise_i.reshape(_N, _H)
    outs = []
    for s in range(_S):
        sl = slice(s * _NT, (s + 1) * _NT)
        item_g, text_g, img_g = _sc_gather(
            p["item_table"], p["text_table"], p["img_table"], ids[sl])
        outs.append(_tc_dense(item_g, text_g, img_g, nt[sl], ni[sl], p))
    out = jnp.concatenate(outs, axis=0)
    return out.reshape(_B, _L, _H)


# speedup vs baseline: 1.0840x; 1.0840x over previous
import functools

import jax
import jax.numpy as jnp
from jax import lax
from jax.experimental import pallas as pl
from jax.experimental.pallas import tpu as pltpu
from jax.experimental.pallas import tpu_sc as plsc

_B, _L, _H, _P, _E = 1024, 50, 128, 512, 4
_N = _B * _L
_NC, _NS = 2, 16
_NW = _NC * _NS
_CH = 40
_BB = 8
_T = _BB * _L


def _sc_gather(item_t, text_t, img_t, ids):
    n = ids.shape[0]
    pw = n // _NW
    nchunk = pw // _CH

    @functools.partial(
        pl.kernel,
        mesh=plsc.VectorSubcoreMesh(core_axis_name="c", subcore_axis_name="s"),
        out_type=(
            jax.ShapeDtypeStruct((n, _H), jnp.float32),
            jax.ShapeDtypeStruct((n, _P), jnp.float32),
            jax.ShapeDtypeStruct((n, _P), jnp.float32),
        ),
        scratch_types=(
            pltpu.VMEM((2, _CH), jnp.int32),
            pltpu.VMEM((2, _CH, _H), jnp.float32),
            pltpu.VMEM((2, _CH, _P), jnp.float32),
            pltpu.VMEM((2, _CH, _P), jnp.float32),
            pltpu.SemaphoreType.DMA,
            pltpu.SemaphoreType.DMA,
            pltpu.SemaphoreType.DMA,
            pltpu.SemaphoreType.DMA,
        ),
    )
    def gather_kernel(item_hbm, text_hbm, img_hbm, ids_hbm,
                      o_item, o_text, o_img,
                      idx_v, buf_h, buf_t, buf_i, g0, g1, w0, w1):
        wid = lax.axis_index("s") * _NC + lax.axis_index("c")
        gsem = (g0, g1)
        wsem = (w0, w1)
        gd = [None, None]
        wd = [None, None]

        def start(k):
            pr = k % 2
            if wd[pr] is not None:
                for d in wd[pr]:
                    d.wait()
                wd[pr] = None
            sl = pl.ds(wid * pw + k * _CH, _CH)
            pltpu.sync_copy(ids_hbm.at[sl], idx_v.at[pr])
            gd[pr] = (
                pltpu.async_copy(item_hbm.at[idx_v.at[pr]], buf_h.at[pr], gsem[pr]),
                pltpu.async_copy(text_hbm.at[idx_v.at[pr]], buf_t.at[pr], gsem[pr]),
                pltpu.async_copy(img_hbm.at[idx_v.at[pr]], buf_i.at[pr], gsem[pr]),
            )

        def finish(k):
            pr = k % 2
            for d in gd[pr]:
                d.wait()
            gd[pr] = None
            sl = pl.ds(wid * pw + k * _CH, _CH)
            wd[pr] = (
                pltpu.async_copy(buf_h.at[pr], o_item.at[sl], wsem[pr]),
                pltpu.async_copy(buf_t.at[pr], o_text.at[sl], wsem[pr]),
                pltpu.async_copy(buf_i.at[pr], o_img.at[sl], wsem[pr]),
            )

        start(0)
        for k in range(1, nchunk):
            start(k)
            finish(k - 1)
        finish(nchunk - 1)
        for pr in (0, 1):
            if wd[pr] is not None:
                for d in wd[pr]:
                    d.wait()

    return gather_kernel(item_t, text_t, img_t, ids)


def _dot(a, b):
    return lax.dot_general(a, b, (((1,), (0,)), ((), ())),
                           preferred_element_type=jnp.float32)


def _tc_body(text_r, img_r, item_r, nt_r, ni_r,
             ftw, ftb, fiw, fib,
             mtw, mtb, stw, stb, miw, mib, siw, sib,
             gw, gb, tew, teb, iew, ieb,
             fw, fb, fg, fbeta, out_r):
    def proj(x, w, b):
        y = _dot(x, w[...]) + b[...]
        nrm = jnp.sqrt(jnp.sum(y * y, axis=-1, keepdims=True))
        return y / jnp.maximum(nrm, 1e-12)

    te = proj(text_r[...], ftw, ftb)
    ie = proj(img_r[...], fiw, fib)

    nt = jnp.concatenate([nt_r[b] for b in range(_BB)], axis=0)
    ni = jnp.concatenate([ni_r[b] for b in range(_BB)], axis=0)

    t_z = _dot(te, mtw[...]) + mtb[...] + jnp.exp(_dot(te, stw[...]) + stb[...]) * nt
    i_z = _dot(ie, miw[...]) + mib[...] + jnp.exp(_dot(ie, siw[...]) + sib[...]) * ni

    jj = lax.broadcasted_iota(jnp.int32, (_E, _E * _H), 0)
    ll = lax.broadcasted_iota(jnp.int32, (_E, _E * _H), 1)
    ee = (jj == (ll >> 7)).astype(jnp.float32)
    neg = jnp.float32(-1e30)

    def moe(z, ewc, ebc):
        logits = _dot(z, gw[...]) + gb[...]
        lt = logits.T
        ii = lax.broadcasted_iota(jnp.int32, (_E, _T), 0)
        m1 = jnp.max(lt, axis=0, keepdims=True)
        a1 = jnp.min(jnp.where(lt == m1, ii, _E), axis=0, keepdims=True)
        msk = jnp.where(ii == a1, neg, lt)
        m2 = jnp.max(msk, axis=0, keepdims=True)
        a2 = jnp.min(jnp.where(msk == m2, ii, _E), axis=0, keepdims=True)
        keep = (ii == a1) | (ii == a2)
        e = jnp.exp(lt - m1)
        w = jnp.where(keep, e, 0.0)
        wn = w / jnp.sum(w, axis=0, keepdims=True)
        gx = lax.dot_general(wn, ee, (((0,), (0,)), ((), ())),
                             preferred_element_type=jnp.float32)
        y = (_dot(z, ewc[...]) + ebc[...]) * gx
        return (y[:, 0:_H] + y[:, _H:2 * _H]
                + y[:, 2 * _H:3 * _H] + y[:, 3 * _H:4 * _H])

    t_out = moe(t_z, tew, teb)
    i_out = moe(i_z, iew, ieb)

    f = _dot(t_out, fw[0]) + _dot(i_out, fw[1]) + fb[...]
    mu = jnp.mean(f, axis=-1, keepdims=True)
    d = f - mu
    v = jnp.mean(d * d, axis=-1, keepdims=True)
    ln = d / jnp.sqrt(v + 1e-5) * fg[...] + fbeta[...]
    res = item_r[...] + jnp.maximum(ln, 0.0)
    for b in range(_BB):
        out_r[b] = res[b * _L:(b + 1) * _L, :]


def _tc_specs_and_args(item_g, text_g, img_g, nt3, ni3, p):
    tok = lambda d: pl.BlockSpec((_T, d), lambda i: (i, 0))
    n3 = pl.BlockSpec((_BB, _L, _H), lambda i: (i, 0, 0))
    full = lambda *shape: pl.BlockSpec(shape, lambda i: (0,) * len(shape))
    r2 = lambda x: x.reshape(1, -1)
    args = (
        text_g, img_g, item_g, nt3, ni3,
        p["fc_text_w"], r2(p["fc_text_b"]), p["fc_img_w"], r2(p["fc_img_b"]),
        p["mu_t_w"], r2(p["mu_t_b"]), p["sg_t_w"], r2(p["sg_t_b"]),
        p["mu_i_w"], r2(p["mu_i_b"]), p["sg_i_w"], r2(p["sg_i_b"]),
        p["gate_w"], r2(p["gate_b"]),
        jnp.transpose(p["te_w"], (1, 0, 2)).reshape(_H, _E * _H),
        p["te_b"].reshape(1, _E * _H),
        jnp.transpose(p["ie_w"], (1, 0, 2)).reshape(_H, _E * _H),
        p["ie_b"].reshape(1, _E * _H),
        p["fus_w"].reshape(2, _H, _H), r2(p["fus_b"]),
        r2(p["fus_ln_g"]), r2(p["fus_ln_b"]),
    )
    in_specs = [
        tok(_P), tok(_P), tok(_H), n3, n3,
        full(_P, _H), full(1, _H), full(_P, _H), full(1, _H),
        full(_H, _H), full(1, _H), full(_H, _H), full(1, _H),
        full(_H, _H), full(1, _H), full(_H, _H), full(1, _H),
        full(_H, _E), full(1, _E),
        full(_H, _E * _H), full(1, _E * _H),
        full(_H, _E * _H), full(1, _E * _H),
        full(2, _H, _H), full(1, _H),
        full(1, _H), full(1, _H),
    ]
    return in_specs, args


def _tc_dense(item_g, text_g, img_g, nt3, ni3, p):
    in_specs, args = _tc_specs_and_args(item_g, text_g, img_g, nt3, ni3, p)
    return pl.pallas_call(
        _tc_body,
        grid=(_B // _BB,),
        in_specs=in_specs,
        out_specs=pl.BlockSpec((_BB, _L, _H), lambda i: (i, 0, 0)),
        out_shape=jax.ShapeDtypeStruct((_B, _L, _H), jnp.float32),
        compiler_params=pltpu.CompilerParams(
            dimension_semantics=("arbitrary",),
        ),
    )(*args)


def kernel(params, noise_t, noise_i, input_ids):
    p = params
    ids = input_ids.reshape(-1).astype(jnp.int32)
    item_g, text_g, img_g = _sc_gather(
        p["item_table"], p["text_table"], p["img_table"], ids)
    return _tc_dense(item_g, text_g, img_g, noise_t, noise_i, p)


# speedup vs baseline: 1.5434x; 1.4237x over previous
import functools

import jax
import jax.numpy as jnp
from jax import lax
from jax.experimental import pallas as pl
from jax.experimental.pallas import tpu as pltpu
from jax.experimental.pallas import tpu_sc as plsc

_B, _L, _H, _P, _E = 1024, 50, 128, 512, 4
_N = _B * _L
_NC, _NS = 2, 16
_NW = _NC * _NS
_CH = 40
_BB = 64
_T = _BB * _L


def _sc_gather(item_t, text_t, img_t, ids):
    n = ids.shape[0]
    pw = n // _NW
    nchunk = pw // _CH

    @functools.partial(
        pl.kernel,
        mesh=plsc.VectorSubcoreMesh(core_axis_name="c", subcore_axis_name="s"),
        out_type=(
            jax.ShapeDtypeStruct((n, _H), jnp.float32),
            jax.ShapeDtypeStruct((n, _P), jnp.float32),
            jax.ShapeDtypeStruct((n, _P), jnp.float32),
        ),
        scratch_types=(
            pltpu.VMEM((2, _CH), jnp.int32),
            pltpu.VMEM((2, _CH, _H), jnp.float32),
            pltpu.VMEM((2, _CH, _P), jnp.float32),
            pltpu.VMEM((2, _CH, _P), jnp.float32),
            pltpu.SemaphoreType.DMA,
            pltpu.SemaphoreType.DMA,
            pltpu.SemaphoreType.DMA,
            pltpu.SemaphoreType.DMA,
        ),
    )
    def gather_kernel(item_hbm, text_hbm, img_hbm, ids_hbm,
                      o_item, o_text, o_img,
                      idx_v, buf_h, buf_t, buf_i, g0, g1, w0, w1):
        wid = lax.axis_index("s") * _NC + lax.axis_index("c")
        gsem = (g0, g1)
        wsem = (w0, w1)
        gd = [None, None]
        wd = [None, None]

        def start(k):
            pr = k % 2
            if wd[pr] is not None:
                for d in wd[pr]:
                    d.wait()
                wd[pr] = None
            sl = pl.ds(wid * pw + k * _CH, _CH)
            pltpu.sync_copy(ids_hbm.at[sl], idx_v.at[pr])
            gd[pr] = (
                pltpu.async_copy(item_hbm.at[idx_v.at[pr]], buf_h.at[pr], gsem[pr]),
                pltpu.async_copy(text_hbm.at[idx_v.at[pr]], buf_t.at[pr], gsem[pr]),
                pltpu.async_copy(img_hbm.at[idx_v.at[pr]], buf_i.at[pr], gsem[pr]),
            )

        def finish(k):
            pr = k % 2
            for d in gd[pr]:
                d.wait()
            gd[pr] = None
            sl = pl.ds(wid * pw + k * _CH, _CH)
            wd[pr] = (
                pltpu.async_copy(buf_h.at[pr], o_item.at[sl], wsem[pr]),
                pltpu.async_copy(buf_t.at[pr], o_text.at[sl], wsem[pr]),
                pltpu.async_copy(buf_i.at[pr], o_img.at[sl], wsem[pr]),
            )

        start(0)
        for k in range(1, nchunk):
            start(k)
            finish(k - 1)
        finish(nchunk - 1)
        for pr in (0, 1):
            if wd[pr] is not None:
                for d in wd[pr]:
                    d.wait()

    return gather_kernel(item_t, text_t, img_t, ids)


def _dot(a, b):
    return lax.dot_general(a, b, (((1,), (0,)), ((), ())),
                           preferred_element_type=jnp.float32)


def _tc_body(text_r, img_r, item_r, nt_r, ni_r,
             ftw, ftb, fiw, fib,
             mst, msb, msi, msbi,
             gw, gb, tew, teb, iew, ieb,
             fw, fb, fg, fbeta, out_r):
    def proj(x, w, b):
        y = _dot(x, w[...]) + b[...]
        ssq = jnp.sum(y * y, axis=-1, keepdims=True)
        return y / jnp.maximum(jnp.sqrt(ssq), 1e-12)

    te = proj(text_r[...], ftw, ftb)
    ie = proj(img_r[...], fiw, fib)

    nt = jnp.concatenate([nt_r[b] for b in range(_BB)], axis=0)
    ni = jnp.concatenate([ni_r[b] for b in range(_BB)], axis=0)

    ms_t = _dot(te, mst[...]) + msb[...]
    t_z = ms_t[:, :_H] + jnp.exp(ms_t[:, _H:]) * nt
    ms_i = _dot(ie, msi[...]) + msbi[...]
    i_z = ms_i[:, :_H] + jnp.exp(ms_i[:, _H:]) * ni

    jj = lax.broadcasted_iota(jnp.int32, (_E, _E * _H), 0)
    ll = lax.broadcasted_iota(jnp.int32, (_E, _E * _H), 1)
    ee = (jj == (ll >> 7)).astype(jnp.float32)
    neg = jnp.float32(-1e30)

    def moe(z, ewc, ebm):
        logits = _dot(z, gw[...]) + gb[...]
        lt = logits.T
        ii = lax.broadcasted_iota(jnp.int32, (_E, _T), 0)
        m1 = jnp.max(lt, axis=0, keepdims=True)
        a1 = jnp.min(jnp.where(lt == m1, ii, _E), axis=0, keepdims=True)
        msk = jnp.where(ii == a1, neg, lt)
        m2 = jnp.max(msk, axis=0, keepdims=True)
        a2 = jnp.min(jnp.where(msk == m2, ii, _E), axis=0, keepdims=True)
        keep = (ii == a1) | (ii == a2)
        e = jnp.exp(lt - m1)
        w = jnp.where(keep, e, 0.0)
        wn = w / jnp.sum(w, axis=0, keepdims=True)
        gx = lax.dot_general(wn, ee, (((0,), (0,)), ((), ())),
                             preferred_element_type=jnp.float32)
        y = _dot(z, ewc[...]) * gx
        eb_mix = lax.dot_general(wn, ebm[...], (((0,), (0,)), ((), ())),
                                 preferred_element_type=jnp.float32)
        return (y[:, 0:_H] + y[:, _H:2 * _H]
                + y[:, 2 * _H:3 * _H] + y[:, 3 * _H:4 * _H] + eb_mix)

    t_out = moe(t_z, tew, teb)
    i_out = moe(i_z, iew, ieb)

    f = _dot(t_out, fw[0]) + _dot(i_out, fw[1]) + fb[...]
    mu = jnp.mean(f, axis=-1, keepdims=True)
    d = f - mu
    v = jnp.mean(d * d, axis=-1, keepdims=True)
    ln = d / jnp.sqrt(v + 1e-5) * fg[...] + fbeta[...]
    res = item_r[...] + jnp.maximum(ln, 0.0)
    for b in range(_BB):
        out_r[b] = res[b * _L:(b + 1) * _L, :]


def _tc_specs_and_args(item_g, text_g, img_g, nt3, ni3, p):
    tok = lambda d: pl.BlockSpec((_T, d), lambda i: (i, 0))
    n3 = pl.BlockSpec((_BB, _L, _H), lambda i: (i, 0, 0))
    full = lambda *shape: pl.BlockSpec(shape, lambda i: (0,) * len(shape))
    r2 = lambda x: x.reshape(1, -1)
    args = (
        text_g, img_g, item_g, nt3, ni3,
        p["fc_text_w"], r2(p["fc_text_b"]), p["fc_img_w"], r2(p["fc_img_b"]),
        jnp.concatenate([p["mu_t_w"], p["sg_t_w"]], axis=1),
        jnp.concatenate([p["mu_t_b"], p["sg_t_b"]]).reshape(1, 2 * _H),
        jnp.concatenate([p["mu_i_w"], p["sg_i_w"]], axis=1),
        jnp.concatenate([p["mu_i_b"], p["sg_i_b"]]).reshape(1, 2 * _H),
        p["gate_w"], r2(p["gate_b"]),
        jnp.transpose(p["te_w"], (1, 0, 2)).reshape(_H, _E * _H),
        p["te_b"],
        jnp.transpose(p["ie_w"], (1, 0, 2)).reshape(_H, _E * _H),
        p["ie_b"],
        p["fus_w"].reshape(2, _H, _H), r2(p["fus_b"]),
        r2(p["fus_ln_g"]), r2(p["fus_ln_b"]),
    )
    in_specs = [
        tok(_P), tok(_P), tok(_H), n3, n3,
        full(_P, _H), full(1, _H), full(_P, _H), full(1, _H),
        full(_H, 2 * _H), full(1, 2 * _H),
        full(_H, 2 * _H), full(1, 2 * _H),
        full(_H, _E), full(1, _E),
        full(_H, _E * _H), full(_E, _H),
        full(_H, _E * _H), full(_E, _H),
        full(2, _H, _H), full(1, _H),
        full(1, _H), full(1, _H),
    ]
    return in_specs, args


def _tc_dense(item_g, text_g, img_g, nt3, ni3, p):
    in_specs, args = _tc_specs_and_args(item_g, text_g, img_g, nt3, ni3, p)
    return pl.pallas_call(
        _tc_body,
        grid=(_B // _BB,),
        in_specs=in_specs,
        out_specs=pl.BlockSpec((_BB, _L, _H), lambda i: (i, 0, 0)),
        out_shape=jax.ShapeDtypeStruct((_B, _L, _H), jnp.float32),
        compiler_params=pltpu.CompilerParams(
            dimension_semantics=("arbitrary",),
        ),
    )(*args)


def kernel(params, noise_t, noise_i, input_ids):
    p = params
    ids = input_ids.reshape(-1).astype(jnp.int32)
    item_g, text_g, img_g = _sc_gather(
        p["item_table"], p["text_table"], p["img_table"], ids)
    return _tc_dense(item_g, text_g, img_g, noise_t, noise_i, p)
